# baseline (speedup 1.0000x reference)
import jax
import jax.numpy as jnp
from jax import lax
from jax.experimental import pallas as pl
from jax.experimental.pallas import tpu as pltpu
from jax.experimental.pallas import tpu_sc as plsc

B = 16384
D = 32
NC = 2
NS = 16
NW = NC * NS
BPW = B // NW
CH = 512
NCH = BPW // CH
NT = 4


def _emb_body(cat_i, sub_i, ele_i, brd_i,
              cat_t, sub_t, ele_t, brd_t,
              out, idx_v, rows_v, sem_i, sem_g, sem_o):
    wid = lax.axis_index("s") * NC + lax.axis_index("c")
    base = wid * BPW
    idxs = (cat_i, sub_i, ele_i, brd_i)
    tabs = (cat_t, sub_t, ele_t, brd_t)
    ic = [
        pltpu.async_copy(idxs[t].at[pl.ds(base, BPW)],
                         idx_v.at[pl.ds(t * BPW, BPW)], sem_i)
        for t in range(NT)
    ]
    for c in ic:
        c.wait()
    gc = []
    for t in range(NT):
        for j in range(NCH):
            o = t * BPW + j * CH
            gc.append(pltpu.async_copy(tabs[t].at[idx_v.at[pl.ds(o, CH)]],
                                       rows_v.at[pl.ds(o, CH)], sem_g))
    oc = []
    for t in range(NT):
        for c in gc[t * NCH:(t + 1) * NCH]:
            c.wait()
        oc.append(pltpu.async_copy(rows_v.at[pl.ds(t * BPW, BPW)],
                                   out.at[pl.ds(t * B + base, BPW)], sem_o))
    for c in oc:
        c.wait()


def kernel(categories, sub_categories, elements, brands,
           category_table, sub_category_table, element_table, brand_table):
    mesh = plsc.VectorSubcoreMesh(core_axis_name="c", subcore_axis_name="s")
    f = pl.kernel(
        _emb_body,
        mesh=mesh,
        out_type=jax.ShapeDtypeStruct((NT * B, D), jnp.float32),
        scratch_types=[
            pltpu.VMEM((NT * BPW,), jnp.int32),
            pltpu.VMEM((NT * BPW, D), jnp.float32),
            pltpu.SemaphoreType.DMA,
            pltpu.SemaphoreType.DMA,
            pltpu.SemaphoreType.DMA,
        ],
        compiler_params=pltpu.CompilerParams(use_tc_tiling_on_sc=False),
    )
    return f(categories, sub_categories, elements, brands,
             category_table, sub_category_table, element_table, brand_table)


# speedup vs baseline: 1.0015x; 1.0015x over previous
import jax
import jax.numpy as jnp
from jax import lax
from jax.experimental import pallas as pl
from jax.experimental.pallas import tpu as pltpu
from jax.experimental.pallas import tpu_sc as plsc

B = 16384
D = 32
NC = 2
NS = 16
NW = NC * NS
BPW = B // NW
NT = 4


def _emb_body(cat_i, sub_i, ele_i, brd_i,
              cat_t, sub_t, ele_t, brd_t,
              out,
              idx0, idx1, idx2, idx3,
              rows0, rows1, rows2, rows3,
              sem_i, sem_g, sem_o):
    wid = lax.axis_index("s") * NC + lax.axis_index("c")
    base = wid * BPW
    idxs = (idx0, idx1, idx2, idx3)
    rows = (rows0, rows1, rows2, rows3)
    tabs = (cat_t, sub_t, ele_t, brd_t)

    ics = [
        pltpu.async_copy(s.at[pl.ds(base, BPW)], idxs[t], sem_i)
        for t, s in enumerate((cat_i, sub_i, ele_i, brd_i))
    ]
    for c in ics:
        c.wait()

    gcs = [
        pltpu.async_copy(tabs[t].at[idxs[t]], rows[t], sem_g)
        for t in range(NT)
    ]
    ocs = []
    for t in range(NT):
        gcs[t].wait()
        ocs.append(pltpu.async_copy(
            rows[t], out.at[pl.ds(t * B + base, BPW)], sem_o))
    for c in ocs:
        c.wait()


def kernel(categories, sub_categories, elements, brands,
           category_table, sub_category_table, element_table, brand_table):
    mesh = plsc.VectorSubcoreMesh(core_axis_name="c", subcore_axis_name="s")
    f = pl.kernel(
        _emb_body,
        mesh=mesh,
        compiler_params=pltpu.CompilerParams(use_tc_tiling_on_sc=False),
        out_type=jax.ShapeDtypeStruct((NT * B, D), jnp.float32),
        scratch_types=(
            [pltpu.VMEM((BPW,), jnp.int32) for _ in range(NT)]
            + [pltpu.VMEM((BPW, D), jnp.float32) for _ in range(NT)]
            + [pltpu.SemaphoreType.DMA,
               pltpu.SemaphoreType.DMA,
               pltpu.SemaphoreType.DMA]
        ),
    )
    return f(categories, sub_categories, elements, brands,
             category_table, sub_category_table, element_table, brand_table)
